# baseline (speedup 1.0000x reference)
import jax
import jax.numpy as jnp
from jax import lax
from jax.experimental import pallas as pl
from jax.experimental.pallas import tpu as pltpu

N = 20000
LANES = 128
ROWS = (N + LANES - 1) // LANES
ROWS = ((ROWS + 7) // 8) * 8
NPAD = ROWS * LANES
MAX_OUT = 200
NMS_THRESH = 0.4
V0 = 0.1
V1 = 0.2
NEG_INF = float("-inf")

NCH = 15


def _argmax_first(v, ix):
    def comb(va, ia, vb, ib):
        keep = (va > vb) | ((va == vb) & (ia < ib))
        return jnp.where(keep, va, vb), jnp.where(keep, ia, ib)

    rows = v.shape[0]
    while rows > 8 and rows % 2 == 0 and rows // 2 >= 8:
        h = rows // 2
        v, ix = comb(v[:h], ix[:h], v[h:], ix[h:])
        rows = h
    if rows > 8:
        e = rows - 8
        v0, i0 = comb(v[:e], ix[:e], v[8:], ix[8:])
        v = jnp.concatenate([v0, v[e:8]], axis=0)
        ix = jnp.concatenate([i0, ix[e:8]], axis=0)
        rows = 8
    while rows > 1:
        h = rows // 2
        v, ix = comb(v[:h], ix[:h], v[h:], ix[h:])
        rows = h
    lanes = v.shape[1]
    while lanes > 1:
        h = lanes // 2
        v, ix = comb(v[:, :h], ix[:, :h], v[:, h:], ix[:, h:])
        lanes = h
    return v, ix


def _nms_body(x_ref, out_ref, chans_ref, s_ref):
    f32 = jnp.float32
    sc = x_ref[0]
    dx = x_ref[1] * f32(V0)
    dy = x_ref[2] * f32(V0)
    dw = x_ref[3] * f32(V1)
    dh = x_ref[4] * f32(V1)
    x_a = x_ref[15]
    y_a = x_ref[16]
    w_a = x_ref[17]
    h_a = x_ref[18]
    xc = dx * w_a + x_a
    yc = dy * h_a + y_a
    w = jnp.exp(dw) * w_a
    h = jnp.exp(dh) * h_a
    ymin = yc - h / 2
    xmin = xc - w / 2
    ymax = yc + h / 2
    xmax = xc + w / 2
    chans_ref[0] = ymin
    chans_ref[1] = xmin
    chans_ref[2] = ymax
    chans_ref[3] = xmax
    for j in range(5):
        chans_ref[4 + 2 * j] = (x_ref[5 + 2 * j] * f32(V0)) * w_a + x_a
        chans_ref[5 + 2 * j] = (x_ref[6 + 2 * j] * f32(V0)) * h_a + y_a
    chans_ref[14] = (ymax - ymin) * (xmax - xmin)
    s_ref[...] = jnp.where(sc >= f32(NMS_THRESH), sc, NEG_INF)

    gid = (lax.broadcasted_iota(jnp.int32, (ROWS, LANES), 0) * LANES
           + lax.broadcasted_iota(jnp.int32, (ROWS, LANES), 1))
    lane_iota = lax.broadcasted_iota(jnp.int32, (1, LANES), 1)
    out_iota = lax.broadcasted_iota(jnp.int32, (1, 16), 1)

    def body(i, carry):
        s = s_ref[...]
        maxv, idxv = _argmax_first(s, gid)
        okv = maxv > NEG_INF
        lonehot = lane_iota == jnp.bitwise_and(idxv, LANES - 1)
        idx = idxv[0, 0]
        r = idx // LANES
        vals = []
        for c in range(14):
            rv = chans_ref[c, pl.ds(r, 1), :]
            vals.append(jnp.sum(jnp.where(lonehot, rv, f32(0.0))))
        sy0, sx0, sy1, sx1 = vals[0], vals[1], vals[2], vals[3]
        area1 = (sy1 - sy0) * (sx1 - sx0)
        iy0 = jnp.maximum(sy0, chans_ref[0])
        ix0 = jnp.maximum(sx0, chans_ref[1])
        iy1 = jnp.minimum(sy1, chans_ref[2])
        ix1 = jnp.minimum(sx1, chans_ref[3])
        inter = (jnp.maximum(iy1 - iy0, f32(0.0))
                 * jnp.maximum(ix1 - ix0, f32(0.0)))
        iou = inter / (area1 + chans_ref[14] - inter + f32(1e-8))
        kill = (iou > f32(NMS_THRESH)) | (gid == idxv)
        s_ref[...] = jnp.where(kill, NEG_INF, s)
        okf = jnp.where(okv, f32(1.0), f32(0.0))
        row = jnp.zeros((1, 16), jnp.float32)
        for c, v in enumerate(vals):
            row = jnp.where(out_iota == c, v, row)
        out_ref[pl.ds(i, 1), :] = row * okf
        return carry

    lax.fori_loop(0, MAX_OUT, body, 0)


def kernel(cls_pred, reg_pred, lnd_pred, anchors):
    scores = cls_pred[0, :, 1]
    x = jnp.concatenate(
        [scores[:, None], reg_pred[0], lnd_pred[0], anchors], axis=1)
    xt = jnp.pad(x.T, ((0, 0), (0, NPAD - N))).reshape(19, ROWS, LANES)
    out = pl.pallas_call(
        _nms_body,
        out_shape=jax.ShapeDtypeStruct((MAX_OUT, 16), jnp.float32),
        scratch_shapes=[
            pltpu.VMEM((NCH, ROWS, LANES), jnp.float32),
            pltpu.VMEM((ROWS, LANES), jnp.float32),
        ],
    )(xt)
    return out[:, :4], out[:, 4:14]


# speedup vs baseline: 1.4731x; 1.4731x over previous
import jax
import jax.numpy as jnp
from jax import lax
from jax.experimental import pallas as pl
from jax.experimental.pallas import tpu as pltpu

N = 20000
LANES = 128
ROWS = (N + LANES - 1) // LANES
ROWS = ((ROWS + 7) // 8) * 8
NPAD = ROWS * LANES
MAX_OUT = 200
NMS_THRESH = 0.4
V0 = 0.1
V1 = 0.2
NEG_INF = float("-inf")
INT_MAX = 2**31 - 1

NCH = 15


def _nms_body(x_ref, out_ref, chans_ref, s_ref):
    f32 = jnp.float32
    sc = x_ref[0]
    dx = x_ref[1] * f32(V0)
    dy = x_ref[2] * f32(V0)
    dw = x_ref[3] * f32(V1)
    dh = x_ref[4] * f32(V1)
    x_a = x_ref[15]
    y_a = x_ref[16]
    w_a = x_ref[17]
    h_a = x_ref[18]
    xc = dx * w_a + x_a
    yc = dy * h_a + y_a
    w = jnp.exp(dw) * w_a
    h = jnp.exp(dh) * h_a
    ymin = yc - h / 2
    xmin = xc - w / 2
    ymax = yc + h / 2
    xmax = xc + w / 2
    chans_ref[0] = ymin
    chans_ref[1] = xmin
    chans_ref[2] = ymax
    chans_ref[3] = xmax
    for j in range(5):
        chans_ref[4 + 2 * j] = (x_ref[5 + 2 * j] * f32(V0)) * w_a + x_a
        chans_ref[5 + 2 * j] = (x_ref[6 + 2 * j] * f32(V0)) * h_a + y_a
    chans_ref[14] = (ymax - ymin) * (xmax - xmin)
    s_ref[...] = jnp.where(sc >= f32(NMS_THRESH), sc, NEG_INF)

    gid = (lax.broadcasted_iota(jnp.int32, (ROWS, LANES), 0) * LANES
           + lax.broadcasted_iota(jnp.int32, (ROWS, LANES), 1))
    lane_iota = lax.broadcasted_iota(jnp.int32, (1, LANES), 1)
    out_iota = lax.broadcasted_iota(jnp.int32, (1, 16), 1)

    def body(i, carry):
        s = s_ref[...]
        m = jnp.max(s)
        ok = m > NEG_INF
        idx = jnp.min(jnp.where(s == m, gid, INT_MAX))
        r = idx // LANES
        lane = idx - r * LANES
        lonehot = lane_iota == lane
        vals = []
        for c in range(14):
            rv = chans_ref[c, pl.ds(r, 1), :]
            vals.append(jnp.sum(jnp.where(lonehot, rv, f32(0.0))))
        sy0, sx0, sy1, sx1 = vals[0], vals[1], vals[2], vals[3]
        area1 = (sy1 - sy0) * (sx1 - sx0)
        iy0 = jnp.maximum(sy0, chans_ref[0])
        ix0 = jnp.maximum(sx0, chans_ref[1])
        iy1 = jnp.minimum(sy1, chans_ref[2])
        ix1 = jnp.minimum(sx1, chans_ref[3])
        inter = (jnp.maximum(iy1 - iy0, f32(0.0))
                 * jnp.maximum(ix1 - ix0, f32(0.0)))
        iou = inter / (area1 + chans_ref[14] - inter + f32(1e-8))
        kill = (iou > f32(NMS_THRESH)) | (gid == idx)
        s_ref[...] = jnp.where(kill, NEG_INF, s)
        okf = jnp.where(ok, f32(1.0), f32(0.0))
        row = jnp.zeros((1, 16), jnp.float32)
        for c, v in enumerate(vals):
            row = jnp.where(out_iota == c, v, row)
        out_ref[pl.ds(i, 1), :] = row * okf
        return carry

    lax.fori_loop(0, MAX_OUT, body, 0, unroll=2)


def kernel(cls_pred, reg_pred, lnd_pred, anchors):
    scores = cls_pred[0, :, 1]
    x = jnp.concatenate(
        [scores[:, None], reg_pred[0], lnd_pred[0], anchors], axis=1)
    xt = jnp.pad(x.T, ((0, 0), (0, NPAD - N))).reshape(19, ROWS, LANES)
    out = pl.pallas_call(
        _nms_body,
        out_shape=jax.ShapeDtypeStruct((MAX_OUT, 16), jnp.float32),
        scratch_shapes=[
            pltpu.VMEM((NCH, ROWS, LANES), jnp.float32),
            pltpu.VMEM((ROWS, LANES), jnp.float32),
        ],
    )(xt)
    return out[:, :4], out[:, 4:14]
